# Initial kernel scaffold; baseline (speedup 1.0000x reference)
import jax
import jax.numpy as jnp
from jax.experimental import pallas as pl


def kernel(x, edge_index, edge_weight, W_ih, W_hh, b_ih, b_hh, W_gcn, b_gcn, W_fc, b_fc):
    raise NotImplementedError("write your pallas kernel here")


# speedup vs baseline: 10.0613x; 10.0613x over previous
import functools

import jax
import jax.numpy as jnp
from jax import lax
from jax.experimental import pallas as pl
from jax.experimental.pallas import tpu as pltpu
from jax.experimental.pallas import tpu_sc as plsc

N, T, D, H, O, E = 10000, 8, 128, 128, 128, 320000
NC, NS = 2, 16
NW = NC * NS
K = 128
EP = 327680
C = EP // (NW * K)
ET = C * K
RPT = N // NS
BL = 2000


def _deg_body(col_hbm, ew_hbm, out_hbm, col_v, ew_v, deg_v):
    cid = lax.axis_index("c")
    sid = lax.axis_index("s")
    wid = sid * NC + cid
    pltpu.sync_copy(col_hbm.at[wid], col_v)
    pltpu.sync_copy(ew_hbm.at[wid], ew_v)

    def zero(i, _):
        deg_v[pl.ds(i * 16, 16)] = jnp.zeros((16,), jnp.float32)
        return 0

    lax.fori_loop(0, N // 16, zero, 0, unroll=8)

    def body(g, _):
        idx = col_v[pl.ds(g * 16, 16)]
        vals = ew_v[pl.ds(g * 16, 16)]
        plsc.addupdate_scatter(deg_v, [idx], vals)
        return 0

    lax.fori_loop(0, ET // 16, body, 0, unroll=8)
    for k in range(N // BL):
        pltpu.sync_copy(deg_v.at[pl.ds(k * BL, BL)], out_hbm.at[k].at[wid])


def _deg_call(col2, ew2):
    mesh = plsc.VectorSubcoreMesh(core_axis_name="c", subcore_axis_name="s", num_cores=NC, num_subcores=NS)
    return pl.kernel(
        _deg_body,
        out_type=jax.ShapeDtypeStruct((N // BL, NW, BL), jnp.float32),
        mesh=mesh,
        scratch_types=[
            pltpu.VMEM((ET,), jnp.int32),
            pltpu.VMEM((ET,), jnp.float32),
            pltpu.VMEM((N,), jnp.float32),
        ],
        compiler_params=pltpu.CompilerParams(needs_layout_passes=False, use_tc_tiling_on_sc=False),
    )(col2, ew2)


def _s_body(row_hbm, col_hbm, ew_hbm, y_hbm, out_hbm,
            row_v, col_v, ew_v, gbuf, s_sh, sem):
    cid = lax.axis_index("c")
    sid = lax.axis_index("s")
    wid = sid * NC + cid
    pltpu.sync_copy(row_hbm.at[wid], row_v)
    pltpu.sync_copy(col_hbm.at[wid], col_v)
    pltpu.sync_copy(ew_hbm.at[wid], ew_v)

    def zg(r, _):
        for s in range(H // 16):
            gbuf[r, pl.ds(s * 16, 16)] = jnp.zeros((16,), jnp.float32)
        return 0

    lax.fori_loop(0, K, zg, 0, unroll=8)
    for k in range(RPT // 125):
        pltpu.sync_copy(gbuf.at[pl.ds(0, 125)],
                        s_sh.at[pl.ds(sid * RPT + k * 125, 125)])
    plsc.subcore_barrier()

    def chunk(j, _):
        pltpu.async_copy(y_hbm.at[row_v.at[j]], gbuf, sem).wait()

        def grp(gidx, _):
            wv16 = ew_v[j, pl.ds(gidx * 16, 16)]
            base = gidx * 16
            for l in range(16):
                wv = jnp.full((16,), wv16[l], jnp.float32)
                for s in range(H // 16):
                    sl = pl.ds(s * 16, 16)
                    gbuf[base + l, sl] = gbuf[base + l, sl] * wv
            return 0

        lax.fori_loop(0, K // 16, grp, 0)
        pltpu.sync_copy(gbuf, s_sh.at[col_v.at[j]], add=True)
        return 0

    lax.fori_loop(0, C, chunk, 0)
    plsc.subcore_barrier()
    for k in range(RPT // 125):
        sl = pl.ds(sid * RPT + k * 125, 125)
        pltpu.sync_copy(s_sh.at[sl], out_hbm.at[cid].at[sl])


def _s_call(row3, col3, ew3, y):
    mesh = plsc.VectorSubcoreMesh(core_axis_name="c", subcore_axis_name="s", num_cores=NC, num_subcores=NS)
    return pl.kernel(
        _s_body,
        out_type=jax.ShapeDtypeStruct((NC, N, H), jnp.float32),
        mesh=mesh,
        scratch_types=[
            pltpu.VMEM((C, K), jnp.int32),
            pltpu.VMEM((C, K), jnp.int32),
            pltpu.VMEM((C, K), jnp.float32),
            pltpu.VMEM((K, H), jnp.float32),
            pltpu.VMEM_SHARED((N, H), jnp.float32),
            pltpu.SemaphoreType.DMA,
        ],
        compiler_params=pltpu.CompilerParams(needs_layout_passes=False, use_tc_tiling_on_sc=False),
    )(row3, col3, ew3, y)


def _lstm_body(x_ref, wih_ref, whh_ref, bias_ref, wgcn_ref, degp_ref,
               xw_ref, y_ref):
    h = jnp.zeros((BL, H), jnp.float32)
    c = jnp.zeros((BL, H), jnp.float32)
    wih = wih_ref[...]
    whh = whh_ref[...]
    bias = bias_ref[...]
    for t in range(T):
        xt = x_ref[:, t, :]
        g = (jnp.dot(xt, wih, preferred_element_type=jnp.float32)
             + jnp.dot(h, whh, preferred_element_type=jnp.float32) + bias)
        i = jax.nn.sigmoid(g[:, 0:H])
        f = jax.nn.sigmoid(g[:, H:2 * H])
        gg = jnp.tanh(g[:, 2 * H:3 * H])
        o = jax.nn.sigmoid(g[:, 3 * H:4 * H])
        c = f * c + i * gg
        h = o * jnp.tanh(c)
    xw = jnp.dot(h, wgcn_ref[...], preferred_element_type=jnp.float32)
    deg = 1.0 + jnp.sum(degp_ref[0], axis=0)
    dis = lax.rsqrt(deg)
    xw_ref[...] = xw
    y_ref[...] = xw * dis[:, None]


def _lstm_call(x, wih_t, whh_t, bias, wgcn_t, deg_part):
    grid = (N // BL,)
    return pl.pallas_call(
        _lstm_body,
        grid=grid,
        in_specs=[
            pl.BlockSpec((BL, T, D), lambda i: (i, 0, 0)),
            pl.BlockSpec((D, 4 * H), lambda i: (0, 0)),
            pl.BlockSpec((H, 4 * H), lambda i: (0, 0)),
            pl.BlockSpec((1, 4 * H), lambda i: (0, 0)),
            pl.BlockSpec((H, H), lambda i: (0, 0)),
            pl.BlockSpec((1, NW, BL), lambda i: (i, 0, 0)),
        ],
        out_specs=[
            pl.BlockSpec((BL, H), lambda i: (i, 0)),
            pl.BlockSpec((BL, H), lambda i: (i, 0)),
        ],
        out_shape=[
            jax.ShapeDtypeStruct((N, H), jnp.float32),
            jax.ShapeDtypeStruct((N, H), jnp.float32),
        ],
    )(x, wih_t, whh_t, bias, wgcn_t, deg_part)


def _final_body(sp_ref, degp_ref, xw_ref, bgcn_ref, wfc_ref, bfc_ref, out_ref):
    s = sp_ref[0] + sp_ref[1]
    deg = 1.0 + jnp.sum(degp_ref[0], axis=0)
    dis = lax.rsqrt(deg)
    xw = xw_ref[...]
    agg = s * dis[:, None] + xw * (dis * dis)[:, None] + bgcn_ref[...]
    act = jnp.maximum(agg, 0.0)
    out_ref[...] = (jnp.dot(act, wfc_ref[...],
                            preferred_element_type=jnp.float32) + bfc_ref[...])


def _final_call(s_part, deg_part, xw, bgcn, wfc_t, bfc):
    grid = (N // BL,)
    return pl.pallas_call(
        _final_body,
        grid=grid,
        in_specs=[
            pl.BlockSpec((NC, BL, H), lambda i: (0, i, 0)),
            pl.BlockSpec((1, NW, BL), lambda i: (i, 0, 0)),
            pl.BlockSpec((BL, H), lambda i: (i, 0)),
            pl.BlockSpec((1, H), lambda i: (0, 0)),
            pl.BlockSpec((H, O), lambda i: (0, 0)),
            pl.BlockSpec((1, O), lambda i: (0, 0)),
        ],
        out_specs=pl.BlockSpec((BL, O), lambda i: (i, 0)),
        out_shape=jax.ShapeDtypeStruct((N, O), jnp.float32),
    )(s_part, deg_part, xw, bgcn, wfc_t, bfc)


def kernel(x, edge_index, edge_weight, W_ih, W_hh, b_ih, b_hh,
           W_gcn, b_gcn, W_fc, b_fc):
    row = edge_index[0]
    col = edge_index[1]
    pad = EP - E
    zi = jnp.zeros((pad,), jnp.int32)
    row_p = jnp.concatenate([row, zi])
    col_p = jnp.concatenate([col, zi])
    ew_p = jnp.concatenate([edge_weight, jnp.zeros((pad,), edge_weight.dtype)])

    deg_part = _deg_call(col_p.reshape(NW, ET), ew_p.reshape(NW, ET))

    bias = (b_ih + b_hh)[None, :]
    xw, y = _lstm_call(x, W_ih.T, W_hh.T, bias, W_gcn.T, deg_part)

    s_part = _s_call(row_p.reshape(NW, C, K), col_p.reshape(NW, C, K),
                     ew_p.reshape(NW, C, K), y)

    return _final_call(s_part, deg_part, xw, b_gcn[None, :], W_fc.T,
                       b_fc[None, :])


# speedup vs baseline: 12.4404x; 1.2365x over previous
import functools

import jax
import jax.numpy as jnp
from jax import lax
from jax.experimental import pallas as pl
from jax.experimental.pallas import tpu as pltpu
from jax.experimental.pallas import tpu_sc as plsc

N, T, D, H, O, E = 10000, 8, 128, 128, 128, 320000
NC, NS = 2, 16
NW = NC * NS
K = 128
EP = 327680
C = EP // (NW * K)
ET = C * K
RPT = N // NS
BL = 2000


def _deg_body(col_hbm, ew_hbm, out_hbm, col_v, ew_v, deg_v):
    cid = lax.axis_index("c")
    sid = lax.axis_index("s")
    wid = sid * NC + cid
    pltpu.sync_copy(col_hbm.at[wid], col_v)
    pltpu.sync_copy(ew_hbm.at[wid], ew_v)

    def zero(i, _):
        deg_v[pl.ds(i * 16, 16)] = jnp.zeros((16,), jnp.float32)
        return 0

    lax.fori_loop(0, N // 16, zero, 0, unroll=8)

    def body(g, _):
        idx = col_v[pl.ds(g * 16, 16)]
        vals = ew_v[pl.ds(g * 16, 16)]
        plsc.addupdate_scatter(deg_v, [idx], vals)
        return 0

    lax.fori_loop(0, ET // 16, body, 0, unroll=8)
    for k in range(N // BL):
        pltpu.sync_copy(deg_v.at[pl.ds(k * BL, BL)], out_hbm.at[k].at[wid])


def _deg_call(col2, ew2):
    mesh = plsc.VectorSubcoreMesh(core_axis_name="c", subcore_axis_name="s", num_cores=NC, num_subcores=NS)
    return pl.kernel(
        _deg_body,
        out_type=jax.ShapeDtypeStruct((N // BL, NW, BL), jnp.float32),
        mesh=mesh,
        scratch_types=[
            pltpu.VMEM((ET,), jnp.int32),
            pltpu.VMEM((ET,), jnp.float32),
            pltpu.VMEM((N,), jnp.float32),
        ],
        compiler_params=pltpu.CompilerParams(needs_layout_passes=False, use_tc_tiling_on_sc=False),
    )(col2, ew2)


NB = 2


def _s_body(row_hbm, cw_hbm, y_hbm, out_hbm,
            row_v, cwb, gb0, gb1, s_sh, sg0, sg1, ss0, ss1, sc0, sc1):
    gbs = (gb0, gb1)
    sgs = (sg0, sg1)
    sss = (ss0, ss1)
    scs = (sc0, sc1)
    cid = lax.axis_index("c")
    sid = lax.axis_index("s")
    wid = sid * NC + cid
    pltpu.sync_copy(row_hbm.at[wid], row_v)

    def zg(r, _):
        for s in range(H // 16):
            gb0[r, pl.ds(s * 16, 16)] = jnp.zeros((16,), jnp.float32)
        return 0

    lax.fori_loop(0, K, zg, 0, unroll=8)
    for k in range(RPT // 125):
        pltpu.sync_copy(gb0.at[pl.ds(0, 125)],
                        s_sh.at[pl.ds(sid * RPT + k * 125, 125)])
    plsc.subcore_barrier()

    def start_gather(j, b):
        pltpu.async_copy(y_hbm.at[row_v.at[j]], gbs[b], sgs[b])

    def wait_gather(j, b):
        pltpu.make_async_copy(y_hbm.at[row_v.at[j]], gbs[b], sgs[b]).wait()

    def start_cw(j, b):
        pltpu.async_copy(cw_hbm.at[wid].at[j], cwb.at[b], scs[b])

    def wait_cw(j, b):
        pltpu.make_async_copy(cw_hbm.at[wid].at[j], cwb.at[b], scs[b]).wait()

    def start_scatter(j, b):
        pltpu.async_copy(gbs[b], s_sh.at[cwb.at[b].at[0]], sss[b], add=True)

    def wait_scatter(j, b):
        pltpu.make_async_copy(gbs[b], s_sh.at[cwb.at[b].at[0]], sss[b]).wait()

    def scale(j, b):
        gb = gbs[b]

        def grp(gidx, _):
            wv16 = plsc.bitcast(cwb[b, 1, pl.ds(gidx * 16, 16)], jnp.float32)
            base = gidx * 16
            for l in range(16):
                wv = jnp.full((16,), wv16[l], jnp.float32)
                for s in range(H // 16):
                    sl = pl.ds(s * 16, 16)
                    gb[base + l, sl] = gb[base + l, sl] * wv
            return 0

        lax.fori_loop(0, K // 16, grp, 0)

    start_cw(0, 0)
    start_gather(0, 0)
    start_cw(1, 1)
    start_gather(1, 1)
    wait_gather(0, 0)
    wait_cw(0, 0)
    scale(0, 0)
    start_scatter(0, 0)

    def mid(jj, _):
        for b in range(NB):
            j = jj * NB - 1 + b
            bb = (b + 1) % NB
            wait_gather(j, bb)
            wait_cw(j, bb)
            wait_scatter(j - 1, b)
            start_gather(j + 1, b)
            start_cw(j + 1, b)
            scale(j, bb)
            start_scatter(j, bb)
        return 0

    lax.fori_loop(1, C // NB, mid, 0)

    wait_gather(C - 1, 1)
    wait_cw(C - 1, 1)
    wait_scatter(C - 2, 0)
    scale(C - 1, 1)
    start_scatter(C - 1, 1)
    wait_scatter(C - 1, 1)

    plsc.subcore_barrier()
    for k in range(RPT // 125):
        sl = pl.ds(sid * RPT + k * 125, 125)
        pltpu.sync_copy(s_sh.at[sl], out_hbm.at[cid].at[sl])


def _s_call(row3, cw4, y):
    mesh = plsc.VectorSubcoreMesh(core_axis_name="c", subcore_axis_name="s", num_cores=NC, num_subcores=NS)
    return pl.kernel(
        _s_body,
        out_type=jax.ShapeDtypeStruct((NC, N, H), jnp.float32),
        mesh=mesh,
        scratch_types=[
            pltpu.VMEM((C, K), jnp.int32),
            pltpu.VMEM((NB, 2, K), jnp.int32),
            pltpu.VMEM((K, H), jnp.float32),
            pltpu.VMEM((K, H), jnp.float32),
            pltpu.VMEM_SHARED((N, H), jnp.float32),
            pltpu.SemaphoreType.DMA,
            pltpu.SemaphoreType.DMA,
            pltpu.SemaphoreType.DMA,
            pltpu.SemaphoreType.DMA,
            pltpu.SemaphoreType.DMA,
            pltpu.SemaphoreType.DMA,
        ],
        compiler_params=pltpu.CompilerParams(needs_layout_passes=False, use_tc_tiling_on_sc=False),
    )(row3, cw4, y)


def _lstm_body(x_ref, wih_ref, whh_ref, bias_ref, wgcn_ref, degp_ref,
               xw_ref, y_ref):
    h = jnp.zeros((BL, H), jnp.float32)
    c = jnp.zeros((BL, H), jnp.float32)
    wih = wih_ref[...]
    whh = whh_ref[...]
    bias = bias_ref[...]
    for t in range(T):
        xt = x_ref[:, t, :]
        g = (jnp.dot(xt, wih, preferred_element_type=jnp.float32)
             + jnp.dot(h, whh, preferred_element_type=jnp.float32) + bias)
        i = jax.nn.sigmoid(g[:, 0:H])
        f = jax.nn.sigmoid(g[:, H:2 * H])
        gg = jnp.tanh(g[:, 2 * H:3 * H])
        o = jax.nn.sigmoid(g[:, 3 * H:4 * H])
        c = f * c + i * gg
        h = o * jnp.tanh(c)
    xw = jnp.dot(h, wgcn_ref[...], preferred_element_type=jnp.float32)
    deg = 1.0 + jnp.sum(degp_ref[0], axis=0)
    dis = lax.rsqrt(deg)
    xw_ref[...] = xw
    y_ref[...] = xw * dis[:, None]


def _lstm_call(x, wih_t, whh_t, bias, wgcn_t, deg_part):
    grid = (N // BL,)
    return pl.pallas_call(
        _lstm_body,
        grid=grid,
        in_specs=[
            pl.BlockSpec((BL, T, D), lambda i: (i, 0, 0)),
            pl.BlockSpec((D, 4 * H), lambda i: (0, 0)),
            pl.BlockSpec((H, 4 * H), lambda i: (0, 0)),
            pl.BlockSpec((1, 4 * H), lambda i: (0, 0)),
            pl.BlockSpec((H, H), lambda i: (0, 0)),
            pl.BlockSpec((1, NW, BL), lambda i: (i, 0, 0)),
        ],
        out_specs=[
            pl.BlockSpec((BL, H), lambda i: (i, 0)),
            pl.BlockSpec((BL, H), lambda i: (i, 0)),
        ],
        out_shape=[
            jax.ShapeDtypeStruct((N, H), jnp.float32),
            jax.ShapeDtypeStruct((N, H), jnp.float32),
        ],
    )(x, wih_t, whh_t, bias, wgcn_t, deg_part)


def _final_body(sp_ref, degp_ref, xw_ref, bgcn_ref, wfc_ref, bfc_ref, out_ref):
    s = sp_ref[0] + sp_ref[1]
    deg = 1.0 + jnp.sum(degp_ref[0], axis=0)
    dis = lax.rsqrt(deg)
    xw = xw_ref[...]
    agg = s * dis[:, None] + xw * (dis * dis)[:, None] + bgcn_ref[...]
    act = jnp.maximum(agg, 0.0)
    out_ref[...] = (jnp.dot(act, wfc_ref[...],
                            preferred_element_type=jnp.float32) + bfc_ref[...])


def _final_call(s_part, deg_part, xw, bgcn, wfc_t, bfc):
    grid = (N // BL,)
    return pl.pallas_call(
        _final_body,
        grid=grid,
        in_specs=[
            pl.BlockSpec((NC, BL, H), lambda i: (0, i, 0)),
            pl.BlockSpec((1, NW, BL), lambda i: (i, 0, 0)),
            pl.BlockSpec((BL, H), lambda i: (i, 0)),
            pl.BlockSpec((1, H), lambda i: (0, 0)),
            pl.BlockSpec((H, O), lambda i: (0, 0)),
            pl.BlockSpec((1, O), lambda i: (0, 0)),
        ],
        out_specs=pl.BlockSpec((BL, O), lambda i: (i, 0)),
        out_shape=jax.ShapeDtypeStruct((N, O), jnp.float32),
    )(s_part, deg_part, xw, bgcn, wfc_t, bfc)


def kernel(x, edge_index, edge_weight, W_ih, W_hh, b_ih, b_hh,
           W_gcn, b_gcn, W_fc, b_fc):
    row = edge_index[0]
    col = edge_index[1]
    pad = EP - E
    zi = jnp.zeros((pad,), jnp.int32)
    row_p = jnp.concatenate([row, zi])
    col_p = jnp.concatenate([col, zi])
    ew_p = jnp.concatenate([edge_weight, jnp.zeros((pad,), edge_weight.dtype)])

    deg_part = _deg_call(col_p.reshape(NW, ET), ew_p.reshape(NW, ET))

    bias = (b_ih + b_hh)[None, :]
    xw, y = _lstm_call(x, W_ih.T, W_hh.T, bias, W_gcn.T, deg_part)

    cw4 = jnp.stack([col_p.reshape(NW, C, K),
                     lax.bitcast_convert_type(ew_p, jnp.int32).reshape(NW, C, K)],
                    axis=2)
    s_part = _s_call(row_p.reshape(NW, C, K), cw4, y)

    return _final_call(s_part, deg_part, xw, b_gcn[None, :], W_fc.T,
                       b_fc[None, :])


# speedup vs baseline: 15.3889x; 1.2370x over previous
import functools

import jax
import jax.numpy as jnp
from jax import lax
from jax.experimental import pallas as pl
from jax.experimental.pallas import tpu as pltpu
from jax.experimental.pallas import tpu_sc as plsc

N, T, D, H, O, E = 10000, 8, 128, 128, 128, 320000
NC, NS = 2, 16
NW = NC * NS
K = 128
EP = 327680
C = EP // (NW * K)
ET = C * K
RPT = N // NS
BL = 2000


def _deg_body(col_hbm, ew_hbm, out_hbm, col_v, ew_v, deg_v):
    cid = lax.axis_index("c")
    sid = lax.axis_index("s")
    wid = sid * NC + cid
    pltpu.sync_copy(col_hbm.at[wid], col_v)
    pltpu.sync_copy(ew_hbm.at[wid], ew_v)

    def zero(i, _):
        deg_v[pl.ds(i * 16, 16)] = jnp.zeros((16,), jnp.float32)
        return 0

    lax.fori_loop(0, N // 16, zero, 0, unroll=8)

    def body(g, _):
        idx = col_v[pl.ds(g * 16, 16)]
        vals = ew_v[pl.ds(g * 16, 16)]
        plsc.addupdate_scatter(deg_v, [idx], vals)
        return 0

    lax.fori_loop(0, ET // 16, body, 0, unroll=8)
    for k in range(N // BL):
        pltpu.sync_copy(deg_v.at[pl.ds(k * BL, BL)], out_hbm.at[k].at[wid])


def _deg_call(col2, ew2):
    mesh = plsc.VectorSubcoreMesh(core_axis_name="c", subcore_axis_name="s", num_cores=NC, num_subcores=NS)
    return pl.kernel(
        _deg_body,
        out_type=jax.ShapeDtypeStruct((N // BL, NW, BL), jnp.float32),
        mesh=mesh,
        scratch_types=[
            pltpu.VMEM((ET,), jnp.int32),
            pltpu.VMEM((ET,), jnp.float32),
            pltpu.VMEM((N,), jnp.float32),
        ],
        compiler_params=pltpu.CompilerParams(needs_layout_passes=False, use_tc_tiling_on_sc=False),
    )(col2, ew2)


NB = 2

def _perm():
    p = [0] * H
    for q in range(H // 32):
        for i in range(16):
            p[32 * q + 2 * i] = 32 * q + i
            p[32 * q + 2 * i + 1] = 32 * q + 16 + i
    return p

PERM = tuple(_perm())


def _s_body(rcw_hbm, y_hbm, out_hbm,
            rcwb, gb0, gb1, sb0, sb1, s_sh,
            sg0, sg1, ss0, ss1, sr0, sr1, sr2):
    gbs = (gb0, gb1)
    sbs = (sb0, sb1)
    sgs = (sg0, sg1)
    sss = (ss0, ss1)
    srs = (sr0, sr1, sr2)
    cid = lax.axis_index("c")
    sid = lax.axis_index("s")
    wid = sid * NC + cid

    def zg(r, _):
        for q in range(H // 16):
            sb0[r, pl.ds(q * 16, 16)] = jnp.zeros((16,), jnp.float32)
        return 0

    lax.fori_loop(0, K, zg, 0, unroll=8)
    for k in range(RPT // 125):
        pltpu.sync_copy(sb0.at[pl.ds(0, 125)],
                        s_sh.at[pl.ds(sid * RPT + k * 125, 125)])
    plsc.subcore_barrier()

    def start_rcw(j, r):
        pltpu.async_copy(rcw_hbm.at[wid].at[j], rcwb.at[r], srs[r])

    def wait_rcw(j, r):
        pltpu.make_async_copy(rcw_hbm.at[wid].at[j], rcwb.at[r],
                              srs[r]).wait()

    def start_gather(j, r, b):
        pltpu.async_copy(y_hbm.at[rcwb.at[r].at[0]], gbs[b], sgs[b])

    def wait_gather(j, r, b):
        pltpu.make_async_copy(y_hbm.at[rcwb.at[r].at[0]], gbs[b],
                              sgs[b]).wait()

    def start_scatter(j, r, b):
        pltpu.async_copy(sbs[b], s_sh.at[rcwb.at[r].at[1]], sss[b], add=True)

    def wait_scatter(j, r, b):
        pltpu.make_async_copy(sbs[b], s_sh.at[rcwb.at[r].at[1]],
                              sss[b]).wait()

    def scale(r, b):
        gb = gbs[b]
        sb = sbs[b]

        def grp(gidx, _):
            wv16 = plsc.bitcast(rcwb[r, 2, pl.ds(gidx * 16, 16)],
                                jnp.float32)
            base = gidx * 16
            for l in range(16):
                wv = jnp.full((16,), wv16[l], jnp.float32)
                for q in range(H // 32):
                    pk = gb[base + l, pl.ds(q * 32, 32)]
                    lo, hi = plsc.unpack(pk,
                                         format=plsc.PackFormat.INTERLEAVED)
                    sb[base + l, pl.ds(q * 32, 16)] = lo * wv
                    sb[base + l, pl.ds(q * 32 + 16, 16)] = hi * wv
            return 0

        lax.fori_loop(0, K // 16, grp, 0)

    start_rcw(0, 0)
    start_rcw(1, 1)
    wait_rcw(0, 0)
    start_gather(0, 0, 0)
    start_rcw(2, 2)
    wait_gather(0, 0, 0)
    wait_rcw(1, 1)
    start_gather(1, 1, 1)
    scale(0, 0)
    start_scatter(0, 0, 0)

    def mid(m, _):
        for k in range(1, 7):
            j = m * 6 + k
            b = k % 2
            r = k % 3
            wait_gather(j, r, b)
            wait_scatter(j - 1, (k - 1) % 3, (k - 1) % 2)
            wait_rcw(j + 1, (k + 1) % 3)
            start_gather(j + 1, (k + 1) % 3, (k + 1) % 2)
            start_rcw(j + 2, (k + 2) % 3)
            scale(r, b)
            start_scatter(j, r, b)
        return 0

    lax.fori_loop(0, (C - 2) // 6, mid, 0)

    wait_gather(C - 1, 1, 1)
    wait_scatter(C - 2, 0, 0)
    scale(1, 1)
    start_scatter(C - 1, 1, 1)
    wait_scatter(C - 1, 1, 1)

    plsc.subcore_barrier()
    for k in range(RPT // 125):
        sl = pl.ds(sid * RPT + k * 125, 125)
        pltpu.sync_copy(s_sh.at[sl], out_hbm.at[cid].at[sl])


def _s_call(rcw4, yb):
    mesh = plsc.VectorSubcoreMesh(core_axis_name="c", subcore_axis_name="s", num_cores=NC, num_subcores=NS)
    return pl.kernel(
        _s_body,
        out_type=jax.ShapeDtypeStruct((NC, N, H), jnp.float32),
        mesh=mesh,
        scratch_types=[
            pltpu.VMEM((3, 3, K), jnp.int32),
            pltpu.VMEM((K, H), jnp.bfloat16),
            pltpu.VMEM((K, H), jnp.bfloat16),
            pltpu.VMEM((K, H), jnp.float32),
            pltpu.VMEM((K, H), jnp.float32),
            pltpu.VMEM_SHARED((N, H), jnp.float32),
            pltpu.SemaphoreType.DMA,
            pltpu.SemaphoreType.DMA,
            pltpu.SemaphoreType.DMA,
            pltpu.SemaphoreType.DMA,
            pltpu.SemaphoreType.DMA,
            pltpu.SemaphoreType.DMA,
            pltpu.SemaphoreType.DMA,
        ],
        compiler_params=pltpu.CompilerParams(needs_layout_passes=False, use_tc_tiling_on_sc=False),
    )(rcw4, yb)


def _lstm_body(x_ref, wih_ref, whh_ref, bias_ref, wgcn_ref, degp_ref,
               xw_ref, y_ref):
    h = jnp.zeros((BL, H), jnp.float32)
    c = jnp.zeros((BL, H), jnp.float32)
    wih = wih_ref[...]
    whh = whh_ref[...]
    bias = bias_ref[...]
    for t in range(T):
        xt = x_ref[:, t, :]
        g = (jnp.dot(xt, wih, preferred_element_type=jnp.float32)
             + jnp.dot(h, whh, preferred_element_type=jnp.float32) + bias)
        i = jax.nn.sigmoid(g[:, 0:H])
        f = jax.nn.sigmoid(g[:, H:2 * H])
        gg = jnp.tanh(g[:, 2 * H:3 * H])
        o = jax.nn.sigmoid(g[:, 3 * H:4 * H])
        c = f * c + i * gg
        h = o * jnp.tanh(c)
    xw = jnp.dot(h, wgcn_ref[...], preferred_element_type=jnp.float32)
    deg = 1.0 + jnp.sum(degp_ref[0], axis=0)
    dis = lax.rsqrt(deg)
    xw_ref[...] = xw
    y_ref[...] = xw * dis[:, None]


def _lstm_call(x, wih_t, whh_t, bias, wgcn_t, deg_part):
    grid = (N // BL,)
    return pl.pallas_call(
        _lstm_body,
        grid=grid,
        in_specs=[
            pl.BlockSpec((BL, T, D), lambda i: (i, 0, 0)),
            pl.BlockSpec((D, 4 * H), lambda i: (0, 0)),
            pl.BlockSpec((H, 4 * H), lambda i: (0, 0)),
            pl.BlockSpec((1, 4 * H), lambda i: (0, 0)),
            pl.BlockSpec((H, H), lambda i: (0, 0)),
            pl.BlockSpec((1, NW, BL), lambda i: (i, 0, 0)),
        ],
        out_specs=[
            pl.BlockSpec((BL, H), lambda i: (i, 0)),
            pl.BlockSpec((BL, H), lambda i: (i, 0)),
        ],
        out_shape=[
            jax.ShapeDtypeStruct((N, H), jnp.float32),
            jax.ShapeDtypeStruct((N, H), jnp.float32),
        ],
    )(x, wih_t, whh_t, bias, wgcn_t, deg_part)


def _final_body(sp_ref, degp_ref, xw_ref, bgcn_ref, wfc_ref, bfc_ref, out_ref):
    s = sp_ref[0] + sp_ref[1]
    deg = 1.0 + jnp.sum(degp_ref[0], axis=0)
    dis = lax.rsqrt(deg)
    xw = xw_ref[...]
    agg = s * dis[:, None] + xw * (dis * dis)[:, None] + bgcn_ref[...]
    act = jnp.maximum(agg, 0.0)
    out_ref[...] = (jnp.dot(act, wfc_ref[...],
                            preferred_element_type=jnp.float32) + bfc_ref[...])


def _final_call(s_part, deg_part, xw, bgcn, wfc_t, bfc):
    grid = (N // BL,)
    return pl.pallas_call(
        _final_body,
        grid=grid,
        in_specs=[
            pl.BlockSpec((NC, BL, H), lambda i: (0, i, 0)),
            pl.BlockSpec((1, NW, BL), lambda i: (i, 0, 0)),
            pl.BlockSpec((BL, H), lambda i: (i, 0)),
            pl.BlockSpec((1, H), lambda i: (0, 0)),
            pl.BlockSpec((H, O), lambda i: (0, 0)),
            pl.BlockSpec((1, O), lambda i: (0, 0)),
        ],
        out_specs=pl.BlockSpec((BL, O), lambda i: (i, 0)),
        out_shape=jax.ShapeDtypeStruct((N, O), jnp.float32),
    )(s_part, deg_part, xw, bgcn, wfc_t, bfc)


def kernel(x, edge_index, edge_weight, W_ih, W_hh, b_ih, b_hh,
           W_gcn, b_gcn, W_fc, b_fc):
    row = edge_index[0]
    col = edge_index[1]
    pad = EP - E
    zi = jnp.zeros((pad,), jnp.int32)
    row_p = jnp.concatenate([row, zi])
    col_p = jnp.concatenate([col, zi])
    ew_p = jnp.concatenate([edge_weight, jnp.zeros((pad,), edge_weight.dtype)])

    deg_part = _deg_call(col_p.reshape(NW, ET), ew_p.reshape(NW, ET))

    bias = (b_ih + b_hh)[None, :]
    xw, y = _lstm_call(x, W_ih.T, W_hh.T, bias, W_gcn.T, deg_part)

    rcw4 = jnp.stack([row_p.reshape(NW, C, K), col_p.reshape(NW, C, K),
                      lax.bitcast_convert_type(ew_p, jnp.int32).reshape(NW, C, K)],
                     axis=2)
    yb = y[:, jnp.array(PERM, jnp.int32)].astype(jnp.bfloat16)
    s_part = _s_call(rcw4, yb)

    return _final_call(s_part, deg_part, xw, b_gcn[None, :], W_fc.T,
                       b_fc[None, :])
